# Initial kernel scaffold; baseline (speedup 1.0000x reference)
import jax
import jax.numpy as jnp
from jax.experimental import pallas as pl


def kernel(m_atoms, m_bonds, m_edges, p_atoms, p_edges, W_m1, b_m1, W_m2, b_m2, W_go, b_go, W_p1, b_p1, W_p2, b_p2, W_gop, b_gop, W_fc1, b_fc1, W_fc3, b_fc3, W_fc2, b_fc2):
    raise NotImplementedError("write your pallas kernel here")


# speedup vs baseline: 3.8867x; 3.8867x over previous
import functools

import jax
import jax.numpy as jnp
from jax.experimental import pallas as pl

B, N, DEG, NP = 64, 64, 6, 512
F_M = 37
F_P = 480


def _dot(a, b):
    return jax.lax.dot_general(a, b, (((1,), (0,)), ((), ())),
                               preferred_element_type=jnp.float32)


def _body(m_atoms_ref, m_bonds_ref, m_edges_ref, p_atoms_ref, p_edges_ref,
          W_m1_ref, b_m1_ref, W_m2_ref, b_m2_ref, W_go_ref, b_go_ref,
          W_p1_ref, b_p1_ref, W_p2_ref, b_p2_ref, W_gop_ref, b_gop_ref,
          fp_m_ref, fp_p_ref):
    a0 = m_atoms_ref[0]
    bonds = m_bonds_ref[0]
    bsum = jnp.sum(bonds, axis=1)
    edges = m_edges_ref[0]

    col = jax.lax.broadcasted_iota(jnp.int32, (N, N), 1)
    G = jnp.zeros((N, N), jnp.float32)
    for d in range(DEG):
        G = G + (edges[:, d][:, None] == col).astype(jnp.float32)

    W_m1 = W_m1_ref[...]
    h = _dot(a0 + _dot(G, a0), W_m1[:F_M]) + _dot(bsum, W_m1[F_M:]) \
        + b_m1_ref[...][None, :]
    h1 = jax.nn.relu(h)

    W_m2 = W_m2_ref[...]
    h = _dot(h1 + _dot(G, h1), W_m2[:128]) + _dot(bsum, W_m2[128:]) \
        + b_m2_ref[...][None, :]
    h2 = jax.nn.relu(h)

    W_go = W_go_ref[...]
    go = jnp.tanh(_dot(h2, W_go[:128]) + _dot(bsum, W_go[128:])
                  + b_go_ref[...][None, :])
    fp_m_ref[0] = jnp.sum(go, axis=0, keepdims=True)

    x = p_atoms_ref[0]
    adj = p_edges_ref[0]
    t1 = _dot(x, W_p1_ref[...])
    p1 = jax.nn.relu(_dot(adj, t1) + b_p1_ref[...][None, :])
    t2 = _dot(p1, W_p2_ref[...])
    p2 = jax.nn.relu(_dot(adj, t2) + b_p2_ref[...][None, :])
    g = jnp.tanh(_dot(p2, W_gop_ref[...]) + b_gop_ref[...][None, :])
    fp_p_ref[0] = jnp.sum(g, axis=0, keepdims=True)


def _head_body(fp_m_ref, fp_p_ref, W_fc1_ref, b_fc1_ref, W_fc3_ref,
               b_fc3_ref, W_fc2_ref, b_fc2_ref, out_ref):
    W_fc1 = W_fc1_ref[...]
    tmp = _dot(fp_m_ref[...], W_fc1[:128]) + _dot(fp_p_ref[...], W_fc1[128:]) \
        + b_fc1_ref[...][None, :]
    tmp1 = _dot(tmp, W_fc3_ref[...]) + b_fc3_ref[...][None, :]
    out_ref[...] = jax.nn.sigmoid(_dot(tmp1, W_fc2_ref[...])
                                  + b_fc2_ref[...][None, :])


@functools.partial(jax.jit, static_argnames=("interpret",))
def kernel(m_atoms, m_bonds, m_edges, p_atoms, p_edges,
           W_m1, b_m1, W_m2, b_m2, W_go, b_go,
           W_p1, b_p1, W_p2, b_p2, W_gop, b_gop,
           W_fc1, b_fc1, W_fc3, b_fc3, W_fc2, b_fc2, interpret=False):
    whole = lambda *s: pl.BlockSpec(s, lambda b: (0,) * len(s))
    per_b3 = lambda d1, d2: pl.BlockSpec((1, d1, d2), lambda b: (b, 0, 0))

    fp_m, fp_p = pl.pallas_call(
        _body,
        grid=(B,),
        in_specs=[
            per_b3(N, F_M),
            pl.BlockSpec((1, N, DEG, 6), lambda b: (b, 0, 0, 0)),
            per_b3(N, DEG),
            per_b3(NP, F_P),
            per_b3(NP, NP),
            whole(43, 128), whole(128),
            whole(134, 128), whole(128),
            whole(134, 128), whole(128),
            whole(F_P, 200), whole(200),
            whole(200, 100), whole(100),
            whole(100, 128), whole(128),
        ],
        out_specs=[pl.BlockSpec((1, 1, 128), lambda b: (b, 0, 0)),
                   pl.BlockSpec((1, 1, 128), lambda b: (b, 0, 0))],
        out_shape=[jax.ShapeDtypeStruct((B, 1, 128), jnp.float32),
                   jax.ShapeDtypeStruct((B, 1, 128), jnp.float32)],
        interpret=interpret,
    )(m_atoms, m_bonds, m_edges, p_atoms, p_edges,
      W_m1, b_m1, W_m2, b_m2, W_go, b_go,
      W_p1, b_p1, W_p2, b_p2, W_gop, b_gop)

    out = pl.pallas_call(
        _head_body,
        out_shape=jax.ShapeDtypeStruct((B, 1), jnp.float32),
        interpret=interpret,
    )(fp_m.reshape(B, 128), fp_p.reshape(B, 128),
      W_fc1, b_fc1, W_fc3, b_fc3, W_fc2, b_fc2)
    return out


# speedup vs baseline: 4.2265x; 1.0874x over previous
import functools

import jax
import jax.numpy as jnp
from jax.experimental import pallas as pl

B, N, DEG, NP = 64, 64, 6, 512
F_M = 37
F_P = 480
BB = 2


def _dot(a, b):
    return jax.lax.dot_general(a, b, (((1,), (0,)), ((), ())),
                               preferred_element_type=jnp.float32)


def _body(m_atoms_ref, m_bonds_ref, m_edges_ref, p_atoms_ref, p_edges_ref,
          W_m1_ref, b_m1_ref, W_m2_ref, b_m2_ref, W_go_ref, b_go_ref,
          W_p1_ref, b_p1_ref, W_p2_ref, b_p2_ref, W_gop_ref, b_gop_ref,
          fp_m_ref, fp_p_ref):
    W_m1 = W_m1_ref[...]
    W_m2 = W_m2_ref[...]
    W_go = W_go_ref[...]
    col = jax.lax.broadcasted_iota(jnp.int32, (N, N), 1)

    for i in range(BB):
        a0 = m_atoms_ref[i]
        bsum = jnp.sum(m_bonds_ref[i], axis=1)
        edges = m_edges_ref[i]

        G = jnp.zeros((N, N), jnp.float32)
        for d in range(DEG):
            G = G + (edges[:, d][:, None] == col).astype(jnp.float32)

        h = _dot(a0 + _dot(G, a0), W_m1[:F_M]) + _dot(bsum, W_m1[F_M:]) \
            + b_m1_ref[...][None, :]
        h1 = jax.nn.relu(h)

        h = _dot(h1 + _dot(G, h1), W_m2[:128]) + _dot(bsum, W_m2[128:]) \
            + b_m2_ref[...][None, :]
        h2 = jax.nn.relu(h)

        go = jnp.tanh(_dot(h2, W_go[:128]) + _dot(bsum, W_go[128:])
                      + b_go_ref[...][None, :])
        fp_m_ref[i] = jnp.sum(go, axis=0, keepdims=True)

        x = p_atoms_ref[i]
        adj = p_edges_ref[i]
        t1 = _dot(x, W_p1_ref[...])
        p1 = jax.nn.relu(_dot(adj, t1) + b_p1_ref[...][None, :])
        t2 = _dot(p1, W_p2_ref[...])
        p2 = jax.nn.relu(_dot(adj, t2) + b_p2_ref[...][None, :])
        g = jnp.tanh(_dot(p2, W_gop_ref[...]) + b_gop_ref[...][None, :])
        fp_p_ref[i] = jnp.sum(g, axis=0, keepdims=True)


def _head_body(fp_m_ref, fp_p_ref, W_fc1_ref, b_fc1_ref, W_fc3_ref,
               b_fc3_ref, W_fc2_ref, b_fc2_ref, out_ref):
    W_fc1 = W_fc1_ref[...]
    tmp = _dot(fp_m_ref[...], W_fc1[:128]) + _dot(fp_p_ref[...], W_fc1[128:]) \
        + b_fc1_ref[...][None, :]
    tmp1 = _dot(tmp, W_fc3_ref[...]) + b_fc3_ref[...][None, :]
    out_ref[...] = jax.nn.sigmoid(_dot(tmp1, W_fc2_ref[...])
                                  + b_fc2_ref[...][None, :])


@functools.partial(jax.jit, static_argnames=("interpret",))
def kernel(m_atoms, m_bonds, m_edges, p_atoms, p_edges,
           W_m1, b_m1, W_m2, b_m2, W_go, b_go,
           W_p1, b_p1, W_p2, b_p2, W_gop, b_gop,
           W_fc1, b_fc1, W_fc3, b_fc3, W_fc2, b_fc2, interpret=False):
    whole = lambda *s: pl.BlockSpec(s, lambda b: (0,) * len(s))
    per_b3 = lambda d1, d2: pl.BlockSpec((BB, d1, d2), lambda b: (b, 0, 0))

    fp_m, fp_p = pl.pallas_call(
        _body,
        grid=(B // BB,),
        in_specs=[
            per_b3(N, F_M),
            pl.BlockSpec((BB, N, DEG, 6), lambda b: (b, 0, 0, 0)),
            per_b3(N, DEG),
            per_b3(NP, F_P),
            per_b3(NP, NP),
            whole(43, 128), whole(128),
            whole(134, 128), whole(128),
            whole(134, 128), whole(128),
            whole(F_P, 200), whole(200),
            whole(200, 100), whole(100),
            whole(100, 128), whole(128),
        ],
        out_specs=[pl.BlockSpec((BB, 1, 128), lambda b: (b, 0, 0)),
                   pl.BlockSpec((BB, 1, 128), lambda b: (b, 0, 0))],
        out_shape=[jax.ShapeDtypeStruct((B, 1, 128), jnp.float32),
                   jax.ShapeDtypeStruct((B, 1, 128), jnp.float32)],
        interpret=interpret,
    )(m_atoms, m_bonds, m_edges, p_atoms, p_edges,
      W_m1, b_m1, W_m2, b_m2, W_go, b_go,
      W_p1, b_p1, W_p2, b_p2, W_gop, b_gop)

    out = pl.pallas_call(
        _head_body,
        out_shape=jax.ShapeDtypeStruct((B, 1), jnp.float32),
        interpret=interpret,
    )(fp_m.reshape(B, 128), fp_p.reshape(B, 128),
      W_fc1, b_fc1, W_fc3, b_fc3, W_fc2, b_fc2)
    return out


# speedup vs baseline: 4.3589x; 1.0313x over previous
import functools

import jax
import jax.numpy as jnp
from jax.experimental import pallas as pl

B, N, DEG, NP = 64, 64, 6, 512
F_M = 37
F_P = 480
BB = 4


def _dot(a, b):
    return jax.lax.dot_general(a, b, (((1,), (0,)), ((), ())),
                               preferred_element_type=jnp.float32)


def _body(m_atoms_ref, m_bonds_ref, m_edges_ref, p_atoms_ref, p_edges_ref,
          W_m1_ref, b_m1_ref, W_m2_ref, b_m2_ref, W_go_ref, b_go_ref,
          W_p1_ref, b_p1_ref, W_p2_ref, b_p2_ref, W_gop_ref, b_gop_ref,
          fp_m_ref, fp_p_ref):
    W_m1 = W_m1_ref[...]
    W_m2 = W_m2_ref[...]
    W_go = W_go_ref[...]
    col = jax.lax.broadcasted_iota(jnp.int32, (N, N), 1)

    for i in range(BB):
        a0 = m_atoms_ref[i]
        bsum = jnp.sum(m_bonds_ref[i], axis=1)
        edges = m_edges_ref[i]

        G = jnp.zeros((N, N), jnp.float32)
        for d in range(DEG):
            G = G + (edges[:, d][:, None] == col).astype(jnp.float32)

        h = _dot(a0 + _dot(G, a0), W_m1[:F_M]) + _dot(bsum, W_m1[F_M:]) \
            + b_m1_ref[...][None, :]
        h1 = jax.nn.relu(h)

        h = _dot(h1 + _dot(G, h1), W_m2[:128]) + _dot(bsum, W_m2[128:]) \
            + b_m2_ref[...][None, :]
        h2 = jax.nn.relu(h)

        go = jnp.tanh(_dot(h2, W_go[:128]) + _dot(bsum, W_go[128:])
                      + b_go_ref[...][None, :])
        fp_m_ref[i] = jnp.sum(go, axis=0, keepdims=True)

        x = p_atoms_ref[i]
        adj = p_edges_ref[i]
        t1 = _dot(x, W_p1_ref[...])
        p1 = jax.nn.relu(_dot(adj, t1) + b_p1_ref[...][None, :])
        t2 = _dot(p1, W_p2_ref[...])
        p2 = jax.nn.relu(_dot(adj, t2) + b_p2_ref[...][None, :])
        g = jnp.tanh(_dot(p2, W_gop_ref[...]) + b_gop_ref[...][None, :])
        fp_p_ref[i] = jnp.sum(g, axis=0, keepdims=True)


def _head_body(fp_m_ref, fp_p_ref, W_fc1_ref, b_fc1_ref, W_fc3_ref,
               b_fc3_ref, W_fc2_ref, b_fc2_ref, out_ref):
    W_fc1 = W_fc1_ref[...]
    tmp = _dot(fp_m_ref[...], W_fc1[:128]) + _dot(fp_p_ref[...], W_fc1[128:]) \
        + b_fc1_ref[...][None, :]
    tmp1 = _dot(tmp, W_fc3_ref[...]) + b_fc3_ref[...][None, :]
    out_ref[...] = jax.nn.sigmoid(_dot(tmp1, W_fc2_ref[...])
                                  + b_fc2_ref[...][None, :])


@functools.partial(jax.jit, static_argnames=("interpret",))
def kernel(m_atoms, m_bonds, m_edges, p_atoms, p_edges,
           W_m1, b_m1, W_m2, b_m2, W_go, b_go,
           W_p1, b_p1, W_p2, b_p2, W_gop, b_gop,
           W_fc1, b_fc1, W_fc3, b_fc3, W_fc2, b_fc2, interpret=False):
    whole = lambda *s: pl.BlockSpec(s, lambda b: (0,) * len(s))
    per_b3 = lambda d1, d2: pl.BlockSpec((BB, d1, d2), lambda b: (b, 0, 0))

    fp_m, fp_p = pl.pallas_call(
        _body,
        grid=(B // BB,),
        in_specs=[
            per_b3(N, F_M),
            pl.BlockSpec((BB, N, DEG, 6), lambda b: (b, 0, 0, 0)),
            per_b3(N, DEG),
            per_b3(NP, F_P),
            per_b3(NP, NP),
            whole(43, 128), whole(128),
            whole(134, 128), whole(128),
            whole(134, 128), whole(128),
            whole(F_P, 200), whole(200),
            whole(200, 100), whole(100),
            whole(100, 128), whole(128),
        ],
        out_specs=[pl.BlockSpec((BB, 1, 128), lambda b: (b, 0, 0)),
                   pl.BlockSpec((BB, 1, 128), lambda b: (b, 0, 0))],
        out_shape=[jax.ShapeDtypeStruct((B, 1, 128), jnp.float32),
                   jax.ShapeDtypeStruct((B, 1, 128), jnp.float32)],
        interpret=interpret,
    )(m_atoms, m_bonds, m_edges, p_atoms, p_edges,
      W_m1, b_m1, W_m2, b_m2, W_go, b_go,
      W_p1, b_p1, W_p2, b_p2, W_gop, b_gop)

    out = pl.pallas_call(
        _head_body,
        out_shape=jax.ShapeDtypeStruct((B, 1), jnp.float32),
        interpret=interpret,
    )(fp_m.reshape(B, 128), fp_p.reshape(B, 128),
      W_fc1, b_fc1, W_fc3, b_fc3, W_fc2, b_fc2)
    return out
